# baseline (speedup 1.0000x reference)
import functools

import jax
import jax.numpy as jnp
from jax import lax
from jax.experimental import pallas as pl
from jax.experimental.pallas import tpu as pltpu
from jax.experimental.pallas import tpu_sc as plsc

_NBUF = 4


def _mesh_and_info():
    info = plsc.get_sparse_core_info()
    nc, ns = info.num_cores, info.num_subcores
    mesh = plsc.VectorSubcoreMesh(core_axis_name="c", subcore_axis_name="s")
    return mesh, nc, ns


@functools.lru_cache(maxsize=None)
def _make_transpose(d, v_main):
    mesh, nc, ns = _mesh_and_info()
    nw = nc * ns
    blk_rows = 128
    assert v_main % blk_rows == 0
    n_blks = v_main // blk_rows
    iters = (n_blks + nw - 1) // nw
    ring = 6
    dbits = d.bit_length() - 1
    assert d == 1 << dbits

    @functools.partial(
        pl.kernel,
        mesh=mesh,
        out_type=jax.ShapeDtypeStruct((v_main * d,), jnp.float32),
        scratch_types=[
            [pltpu.VMEM((d, blk_rows), jnp.float32)] * ring,
            [pltpu.VMEM((blk_rows * d,), jnp.float32)] * ring,
            [pltpu.SemaphoreType.DMA] * ring,
            [pltpu.SemaphoreType.DMA] * ring,
        ],
        compiler_params=pltpu.CompilerParams(
            use_tc_tiling_on_sc=True, needs_layout_passes=False
        ),
    )
    def transpose_kernel(tt_hbm, out_hbm, in_v, out_v, isems, osems):
        wid = lax.axis_index("s") * nc + lax.axis_index("c")
        lane = lax.iota(jnp.int32, 16)

        def in_desc(i, b):
            blk = wid + i * nw
            return pltpu.make_async_copy(
                tt_hbm.at[:, pl.ds(blk * blk_rows, blk_rows)], in_v[b], isems[b]
            )

        def out_desc(i, b):
            blk = wid + i * nw
            return pltpu.make_async_copy(
                out_v[b],
                out_hbm.at[pl.ds(blk * blk_rows * d, blk_rows * d)],
                osems[b],
            )

        cbn = d // 16

        def transpose(b):
            @plsc.parallel_loop(0, (blk_rows // 16) * cbn, 1, unroll=8)
            def _(k):
                lg = k >> (cbn.bit_length() - 1)
                cb = k & (cbn - 1)
                r_vec = lg * 16 + lane
                rd = r_vec * d + cb * 16
                for c0 in range(16):
                    cperm = (lane + c0) & 15
                    c_vec = cperm + cb * 16
                    vals = plsc.load_gather(in_v[b], [c_vec, r_vec])
                    plsc.store_scatter(out_v[b], [rd + cperm], vals)

        for b in range(ring):

            @pl.when(wid + b * nw < n_blks)
            def _(b=b):
                in_desc(b, b).start()

        def step(i, b):
            blk = wid + i * nw

            @pl.when(blk < n_blks)
            def _():
                @pl.when(i >= ring)
                def _():
                    out_desc(i - ring, b).wait()

                in_desc(i, b).wait()
                transpose(b)
                out_desc(i, b).start()

                @pl.when(blk + ring * nw < n_blks)
                def _():
                    in_desc(i + ring, b).start()

        def body(g, carry):
            for b in range(ring):
                step(g * ring + b, b)
            return carry

        lax.fori_loop(0, (iters + ring - 1) // ring, body, 0)

        i_max = (n_blks - 1 - wid) // nw
        for b in range(ring):
            i_b = i_max - lax.rem(i_max - b + 2 * ring, ring)

            @pl.when(i_b >= 0)
            def _(b=b, i_b=i_b):
                out_desc(i_b, b).wait()

    return transpose_kernel


@functools.lru_cache(maxsize=None)
def _make_gather(n_rows, d, v_main, n_tail, chunk):
    mesh, nc, ns = _mesh_and_info()
    nw = nc * ns
    assert n_rows % nw == 0
    b_per_w = n_rows // nw
    assert b_per_w % chunk == 0 and chunk % 16 == 0
    n_chunks = b_per_w // chunk
    assert n_chunks % _NBUF == 0 and n_chunks >= 2 * _NBUF
    n_groups = n_chunks // _NBUF

    @functools.partial(
        pl.kernel,
        mesh=mesh,
        out_type=jax.ShapeDtypeStruct((n_rows, d), jnp.float32),
        scratch_types=[
            pltpu.VMEM((b_per_w,), jnp.int32),
            pltpu.VMEM((b_per_w,), jnp.int32),
            pltpu.VMEM((n_tail, d), jnp.float32),
            pltpu.VMEM((_NBUF, chunk, d), jnp.float32),
            [pltpu.SemaphoreType.DMA] * _NBUF,
            [pltpu.SemaphoreType.DMA] * _NBUF,
        ],
        compiler_params=pltpu.CompilerParams(
            use_tc_tiling_on_sc=False, needs_layout_passes=False
        ),
    )
    def gather_kernel(
        idx_hbm, table_hbm, tail_hbm, out_hbm, idx_v, idx_c, tail_v, rows_v,
        gsems, ssems,
    ):
        wid = lax.axis_index("s") * nc + lax.axis_index("c")
        base = wid * b_per_w
        pltpu.sync_copy(idx_hbm.at[pl.ds(base, b_per_w)], idx_v)
        pltpu.sync_copy(tail_hbm, tail_v)

        def clamp(g, carry):
            x = idx_v[pl.ds(g * 16, 16)]
            idx_c[pl.ds(g * 16, 16)] = jnp.minimum(x, v_main - 1)
            return carry

        lax.fori_loop(0, b_per_w // 16, clamp, 0)

        def g_desc(i, b):
            return pltpu.make_async_copy(
                table_hbm.at[idx_c.at[pl.ds(i * chunk, chunk)]],
                rows_v.at[b],
                gsems[b],
            )

        def s_desc(i, b):
            return pltpu.make_async_copy(
                rows_v.at[b],
                out_hbm.at[pl.ds(base + i * chunk, chunk)],
                ssems[b],
            )

        def patch_tail(j, b):
            off = j * chunk

            def has_tail(g, acc):
                x = idx_v[pl.ds(off + g * 16, 16)]
                return jnp.maximum(acc, jnp.max(x))

            mx = lax.fori_loop(0, chunk // 16, has_tail, jnp.int32(0))

            @pl.when(mx >= v_main)
            def _():
                lane = lax.iota(jnp.int32, 16)

                def fix(g, carry):
                    idx16 = idx_v[pl.ds(off + g * 16, 16)]
                    m = idx16 >= v_main
                    r = jnp.maximum(idx16 - v_main, 0)
                    jrow = lane + g * 16
                    for c in range(d):
                        cvec = jnp.full((16,), c, jnp.int32)
                        vals = plsc.load_gather(tail_v, [r, cvec], mask=m)
                        plsc.store_scatter(
                            rows_v.at[b], [jrow, cvec], vals, mask=m
                        )
                    return carry

                lax.fori_loop(0, chunk // 16, fix, 0)

        def step(j, b, wait_store, start_gather):
            if wait_store:
                s_desc(j - 2, (b + 2) % _NBUF).wait()
            if start_gather:
                g_desc(j + 2, (b + 2) % _NBUF).start()
            g_desc(j, b).wait()
            patch_tail(j, b)
            s_desc(j, b).start()

        g_desc(0, 0).start()
        g_desc(1, 1).start()
        for b in range(_NBUF):
            step(b, b, wait_store=b >= 2, start_gather=True)

        def body(g, carry):
            for b in range(_NBUF):
                step(g * _NBUF + b, b, wait_store=True, start_gather=True)
            return carry

        lax.fori_loop(1, n_groups - 1, body, 0)

        for b in range(_NBUF):
            j = (n_groups - 1) * _NBUF + b
            step(j, b, wait_store=True, start_gather=j + 2 < n_chunks)
        s_desc(n_chunks - 2, (n_chunks - 2) % _NBUF).wait()
        s_desc(n_chunks - 1, (n_chunks - 1) % _NBUF).wait()

    return gather_kernel


@functools.lru_cache(maxsize=None)
def _make_imagize(bsz, hsz, d):
    mesh, nc, ns = _mesh_and_info()
    nw = nc * ns
    nbt = bsz // 128
    n_units = hsz * nbt
    assert n_units % nw == 0
    iters = n_units // nw
    ring = 3
    unit = 128 * d
    cbn = d // 16

    @functools.partial(
        pl.kernel,
        mesh=mesh,
        out_type=jax.ShapeDtypeStruct((bsz * hsz * d,), jnp.float32),
        scratch_types=[
            [pltpu.VMEM((128, 1, d), jnp.float32)] * ring,
            [pltpu.VMEM((unit,), jnp.float32)] * ring,
            [pltpu.SemaphoreType.DMA] * ring,
            [pltpu.SemaphoreType.DMA] * ring,
        ],
        compiler_params=pltpu.CompilerParams(
            use_tc_tiling_on_sc=False, needs_layout_passes=False
        ),
    )
    def imagize_kernel(in_hbm, out_hbm, in_v, out_v, isems, osems):
        wid = lax.axis_index("s") * nc + lax.axis_index("c")
        lane = lax.iota(jnp.int32, 16)
        zero = jnp.full((16,), 0, jnp.int32)

        def in_desc(i, b):
            u = wid + i * nw
            h = u // nbt
            bt = u - h * nbt
            return pltpu.make_async_copy(
                in_hbm.at[pl.ds(bt * 128, 128), pl.ds(h, 1)], in_v[b], isems[b]
            )

        def start_out(i, b):
            u = wid + i * nw
            h = u // nbt
            bt = u - h * nbt
            for ct in range(d // 8):
                pltpu.make_async_copy(
                    out_v[b].at[pl.ds(ct * 1024, 1024)],
                    out_hbm.at[pl.ds((((h * 8 + ct) * nbt) + bt) * 1024, 1024)],
                    osems[b],
                ).start()

        def wait_out(i, b):
            u = wid + i * nw
            h = u // nbt
            bt = u - h * nbt
            for ct in range(d // 8):
                pltpu.make_async_copy(
                    out_v[b].at[pl.ds(ct * 1024, 1024)],
                    out_hbm.at[pl.ds((((h * 8 + ct) * nbt) + bt) * 1024, 1024)],
                    osems[b],
                ).wait()

        def rearrange(b):
            @plsc.parallel_loop(0, 8 * cbn, 1, unroll=8)
            def _(k):
                bg = k >> (cbn.bit_length() - 1)
                cb = k & (cbn - 1)
                bi = bg * 16 + lane
                sb = cb * 2048
                for c0 in range(16):
                    cperm = (lane + c0) & 15
                    c_vec = cperm + cb * 16
                    dstp = ((cperm >> 3) * 1024 + (cperm & 7) * 128) + sb
                    vals = plsc.load_gather(in_v[b], [bi, zero, c_vec])
                    plsc.store_scatter(out_v[b], [dstp + bi], vals)

        for b in range(ring):

            @pl.when(wid + b * nw < n_units)
            def _(b=b):
                in_desc(b, b).start()

        def step(i, b, dyn):
            if dyn:

                @pl.when(i >= ring)
                def _():
                    wait_out(i - ring, b)

            elif i >= ring:
                wait_out(i - ring, b)

            in_desc(i, b).wait()
            rearrange(b)
            start_out(i, b)

            if dyn:

                @pl.when(i + ring < iters)
                def _():
                    in_desc(i + ring, b).start()

            elif i + ring < iters:
                in_desc(i + ring, b).start()

        def body(g, carry):
            for b in range(ring):
                step(g * ring + b, b, dyn=True)
            return carry

        lax.fori_loop(0, iters // ring, body, 0)
        for j in range(iters - iters % ring, iters):
            step(j, j % ring, dyn=False)
        for j in range(iters - ring, iters):
            wait_out(j, j % ring)

    return imagize_kernel


def kernel(indices, table):
    b, h = indices.shape
    v, d = table.shape
    v_main = (v // 128) * 128
    n_tail = v - v_main
    idx_flat = indices.reshape(b * h).astype(jnp.int32)
    lin = _make_transpose(d, v_main)(table.T)
    tail = lax.slice(table, (v_main, 0), (v, d))
    out = _make_gather(b * h, d, v_main, n_tail, 320)(
        idx_flat, lin.reshape(v_main, d), tail
    )
    img = _make_imagize(b, h, d)(out.reshape(b, h, d))
    x = img.reshape(h, d // 8, b // 128, 8, 128)
    return x.transpose(2, 4, 0, 1, 3).reshape(b, h, d)


# speedup vs baseline: 1.2744x; 1.2744x over previous
import functools

import jax
import jax.numpy as jnp
from jax import lax
from jax.experimental import pallas as pl
from jax.experimental.pallas import tpu as pltpu
from jax.experimental.pallas import tpu_sc as plsc

_NBUF = 4


def _mesh_and_info():
    info = plsc.get_sparse_core_info()
    nc, ns = info.num_cores, info.num_subcores
    mesh = plsc.VectorSubcoreMesh(core_axis_name="c", subcore_axis_name="s")
    return mesh, nc, ns


@functools.lru_cache(maxsize=None)
def _make_transpose(d, v_main):
    mesh, nc, ns = _mesh_and_info()
    nw = nc * ns
    blk_rows = 256
    assert v_main % blk_rows == 0
    n_blks = v_main // blk_rows
    iters = (n_blks + nw - 1) // nw
    ring = 3
    dbits = d.bit_length() - 1
    assert d == 1 << dbits

    @functools.partial(
        pl.kernel,
        mesh=mesh,
        out_type=jax.ShapeDtypeStruct((v_main * d,), jnp.float32),
        scratch_types=[
            [pltpu.VMEM((d, blk_rows), jnp.float32)] * ring,
            [pltpu.VMEM((blk_rows * d,), jnp.float32)] * ring,
            [pltpu.SemaphoreType.DMA] * ring,
            [pltpu.SemaphoreType.DMA] * ring,
        ],
        compiler_params=pltpu.CompilerParams(
            use_tc_tiling_on_sc=True, needs_layout_passes=False
        ),
    )
    def transpose_kernel(tt_hbm, out_hbm, in_v, out_v, isems, osems):
        wid = lax.axis_index("s") * nc + lax.axis_index("c")
        lane = lax.iota(jnp.int32, 16)

        def in_desc(i, b):
            blk = wid + i * nw
            return pltpu.make_async_copy(
                tt_hbm.at[:, pl.ds(blk * blk_rows, blk_rows)], in_v[b], isems[b]
            )

        def out_desc(i, b):
            blk = wid + i * nw
            return pltpu.make_async_copy(
                out_v[b],
                out_hbm.at[pl.ds(blk * blk_rows * d, blk_rows * d)],
                osems[b],
            )

        cbn = d // 16

        def transpose(b):
            @plsc.parallel_loop(0, (blk_rows // 16) * cbn, 1, unroll=8)
            def _(k):
                lg = k >> (cbn.bit_length() - 1)
                cb = k & (cbn - 1)
                r_vec = lg * 16 + lane
                rd = r_vec * d + cb * 16
                for c0 in range(16):
                    cperm = (lane + c0) & 15
                    c_vec = cperm + cb * 16
                    vals = plsc.load_gather(in_v[b], [c_vec, r_vec])
                    plsc.store_scatter(out_v[b], [rd + cperm], vals)

        for b in range(ring):

            @pl.when(wid + b * nw < n_blks)
            def _(b=b):
                in_desc(b, b).start()

        def step(i, b):
            blk = wid + i * nw

            @pl.when(blk < n_blks)
            def _():
                @pl.when(i >= ring)
                def _():
                    out_desc(i - ring, b).wait()

                in_desc(i, b).wait()
                transpose(b)
                out_desc(i, b).start()

                @pl.when(blk + ring * nw < n_blks)
                def _():
                    in_desc(i + ring, b).start()

        def body(g, carry):
            for b in range(ring):
                step(g * ring + b, b)
            return carry

        lax.fori_loop(0, (iters + ring - 1) // ring, body, 0)

        i_max = (n_blks - 1 - wid) // nw
        for b in range(ring):
            i_b = i_max - lax.rem(i_max - b + 2 * ring, ring)

            @pl.when(i_b >= 0)
            def _(b=b, i_b=i_b):
                out_desc(i_b, b).wait()

    return transpose_kernel


@functools.lru_cache(maxsize=None)
def _make_gather(n_rows, d, v_main, n_tail, chunk):
    mesh, nc, ns = _mesh_and_info()
    nw = nc * ns
    assert n_rows % nw == 0
    b_per_w = n_rows // nw
    assert b_per_w % chunk == 0 and chunk % 16 == 0
    n_chunks = b_per_w // chunk
    assert n_chunks % _NBUF == 0 and n_chunks >= 2 * _NBUF
    n_groups = n_chunks // _NBUF

    @functools.partial(
        pl.kernel,
        mesh=mesh,
        out_type=jax.ShapeDtypeStruct((n_rows, d), jnp.float32),
        scratch_types=[
            pltpu.VMEM((b_per_w,), jnp.int32),
            pltpu.VMEM((b_per_w,), jnp.int32),
            pltpu.VMEM((n_tail, d), jnp.float32),
            pltpu.VMEM((_NBUF, chunk, d), jnp.float32),
            [pltpu.SemaphoreType.DMA] * _NBUF,
            [pltpu.SemaphoreType.DMA] * _NBUF,
        ],
        compiler_params=pltpu.CompilerParams(
            use_tc_tiling_on_sc=False, needs_layout_passes=False
        ),
    )
    def gather_kernel(
        idx_hbm, table_hbm, tail_hbm, out_hbm, idx_v, idx_c, tail_v, rows_v,
        gsems, ssems,
    ):
        wid = lax.axis_index("s") * nc + lax.axis_index("c")
        base = wid * b_per_w
        pltpu.sync_copy(idx_hbm.at[pl.ds(base, b_per_w)], idx_v)
        pltpu.sync_copy(tail_hbm, tail_v)

        def clamp(g, carry):
            x = idx_v[pl.ds(g * 16, 16)]
            idx_c[pl.ds(g * 16, 16)] = jnp.minimum(x, v_main - 1)
            return carry

        lax.fori_loop(0, b_per_w // 16, clamp, 0)

        def g_desc(i, b):
            return pltpu.make_async_copy(
                table_hbm.at[idx_c.at[pl.ds(i * chunk, chunk)]],
                rows_v.at[b],
                gsems[b],
            )

        def s_desc(i, b):
            return pltpu.make_async_copy(
                rows_v.at[b],
                out_hbm.at[pl.ds(base + i * chunk, chunk)],
                ssems[b],
            )

        def patch_tail(j, b):
            off = j * chunk

            def has_tail(g, acc):
                x = idx_v[pl.ds(off + g * 16, 16)]
                return jnp.maximum(acc, jnp.max(x))

            mx = lax.fori_loop(0, chunk // 16, has_tail, jnp.int32(0))

            @pl.when(mx >= v_main)
            def _():
                lane = lax.iota(jnp.int32, 16)

                def fix(g, carry):
                    idx16 = idx_v[pl.ds(off + g * 16, 16)]
                    m = idx16 >= v_main
                    r = jnp.maximum(idx16 - v_main, 0)
                    jrow = lane + g * 16
                    for c in range(d):
                        cvec = jnp.full((16,), c, jnp.int32)
                        vals = plsc.load_gather(tail_v, [r, cvec], mask=m)
                        plsc.store_scatter(
                            rows_v.at[b], [jrow, cvec], vals, mask=m
                        )
                    return carry

                lax.fori_loop(0, chunk // 16, fix, 0)

        def step(j, b, wait_store, start_gather):
            if wait_store:
                s_desc(j - 2, (b + 2) % _NBUF).wait()
            if start_gather:
                g_desc(j + 2, (b + 2) % _NBUF).start()
            g_desc(j, b).wait()
            patch_tail(j, b)
            s_desc(j, b).start()

        g_desc(0, 0).start()
        g_desc(1, 1).start()
        for b in range(_NBUF):
            step(b, b, wait_store=b >= 2, start_gather=True)

        def body(g, carry):
            for b in range(_NBUF):
                step(g * _NBUF + b, b, wait_store=True, start_gather=True)
            return carry

        lax.fori_loop(1, n_groups - 1, body, 0)

        for b in range(_NBUF):
            j = (n_groups - 1) * _NBUF + b
            step(j, b, wait_store=True, start_gather=j + 2 < n_chunks)
        s_desc(n_chunks - 2, (n_chunks - 2) % _NBUF).wait()
        s_desc(n_chunks - 1, (n_chunks - 1) % _NBUF).wait()

    return gather_kernel


@functools.lru_cache(maxsize=None)
def _make_imagize(bsz, hsz, d):
    mesh, nc, ns = _mesh_and_info()
    nw = nc * ns
    nbt = bsz // 128
    n_units = hsz * nbt
    assert n_units % nw == 0
    iters = n_units // nw
    ring = 3
    unit = 128 * d
    cbn = d // 16

    @functools.partial(
        pl.kernel,
        mesh=mesh,
        out_type=jax.ShapeDtypeStruct((bsz * hsz * d,), jnp.float32),
        scratch_types=[
            [pltpu.VMEM((128, 1, d), jnp.float32)] * ring,
            [pltpu.VMEM((unit,), jnp.float32)] * ring,
            [pltpu.SemaphoreType.DMA] * ring,
            [pltpu.SemaphoreType.DMA] * ring,
        ],
        compiler_params=pltpu.CompilerParams(
            use_tc_tiling_on_sc=False, needs_layout_passes=False
        ),
    )
    def imagize_kernel(in_hbm, out_hbm, in_v, out_v, isems, osems):
        wid = lax.axis_index("s") * nc + lax.axis_index("c")
        lane = lax.iota(jnp.int32, 16)
        zero = jnp.full((16,), 0, jnp.int32)

        def in_desc(i, b):
            u = wid + i * nw
            h = u // nbt
            bt = u - h * nbt
            return pltpu.make_async_copy(
                in_hbm.at[pl.ds(bt * 128, 128), pl.ds(h, 1)], in_v[b], isems[b]
            )

        def start_out(i, b):
            u = wid + i * nw
            h = u // nbt
            bt = u - h * nbt
            for ct in range(d // 8):
                pltpu.make_async_copy(
                    out_v[b].at[pl.ds(ct * 1024, 1024)],
                    out_hbm.at[pl.ds((((h * 8 + ct) * nbt) + bt) * 1024, 1024)],
                    osems[b],
                ).start()

        def wait_out(i, b):
            u = wid + i * nw
            h = u // nbt
            bt = u - h * nbt
            for ct in range(d // 8):
                pltpu.make_async_copy(
                    out_v[b].at[pl.ds(ct * 1024, 1024)],
                    out_hbm.at[pl.ds((((h * 8 + ct) * nbt) + bt) * 1024, 1024)],
                    osems[b],
                ).wait()

        def rearrange(b):
            @plsc.parallel_loop(0, 8 * cbn, 1, unroll=8)
            def _(k):
                bg = k >> (cbn.bit_length() - 1)
                cb = k & (cbn - 1)
                bi = bg * 16 + lane
                sb = cb * 2048
                for c0 in range(16):
                    cperm = (lane + c0) & 15
                    c_vec = cperm + cb * 16
                    dstp = ((cperm >> 3) * 1024 + (cperm & 7) * 128) + sb
                    vals = plsc.load_gather(in_v[b], [bi, zero, c_vec])
                    plsc.store_scatter(out_v[b], [dstp + bi], vals)

        for b in range(ring):

            @pl.when(wid + b * nw < n_units)
            def _(b=b):
                in_desc(b, b).start()

        def step(i, b, dyn):
            if dyn:

                @pl.when(i >= ring)
                def _():
                    wait_out(i - ring, b)

            elif i >= ring:
                wait_out(i - ring, b)

            in_desc(i, b).wait()
            rearrange(b)
            start_out(i, b)

            if dyn:

                @pl.when(i + ring < iters)
                def _():
                    in_desc(i + ring, b).start()

            elif i + ring < iters:
                in_desc(i + ring, b).start()

        def body(g, carry):
            for b in range(ring):
                step(g * ring + b, b, dyn=True)
            return carry

        lax.fori_loop(0, iters // ring, body, 0)
        for j in range(iters - iters % ring, iters):
            step(j, j % ring, dyn=False)
        for j in range(iters - ring, iters):
            wait_out(j, j % ring)

    return imagize_kernel


def kernel(indices, table):
    b, h = indices.shape
    v, d = table.shape
    v_main = (v // 128) * 128
    n_tail = v - v_main
    idx_flat = indices.reshape(b * h).astype(jnp.int32)
    lin = _make_transpose(d, v_main)(table.T)
    tail = lax.slice(table, (v_main, 0), (v, d))
    out = _make_gather(b * h, d, v_main, n_tail, 320)(
        idx_flat, lin.reshape(v_main, d), tail
    )
    img = _make_imagize(b, h, d)(out.reshape(b, h, d))
    x = img.reshape(h, d // 8, b // 128, 8, 128)
    return x.transpose(2, 4, 0, 1, 3).reshape(b, h, d)


# speedup vs baseline: 1.2817x; 1.0057x over previous
import functools

import jax
import jax.numpy as jnp
from jax import lax
from jax.experimental import pallas as pl
from jax.experimental.pallas import tpu as pltpu
from jax.experimental.pallas import tpu_sc as plsc

_NBUF = 4


def _mesh_and_info():
    info = plsc.get_sparse_core_info()
    nc, ns = info.num_cores, info.num_subcores
    mesh = plsc.VectorSubcoreMesh(core_axis_name="c", subcore_axis_name="s")
    return mesh, nc, ns


@functools.lru_cache(maxsize=None)
def _make_transpose(d, v_main):
    mesh, nc, ns = _mesh_and_info()
    nw = nc * ns
    blk_rows = 384
    assert v_main % blk_rows == 0
    n_blks = v_main // blk_rows
    iters = (n_blks + nw - 1) // nw
    ring = 2
    dbits = d.bit_length() - 1
    assert d == 1 << dbits

    @functools.partial(
        pl.kernel,
        mesh=mesh,
        out_type=jax.ShapeDtypeStruct((v_main * d,), jnp.float32),
        scratch_types=[
            [pltpu.VMEM((d, blk_rows), jnp.float32)] * ring,
            [pltpu.VMEM((blk_rows * d,), jnp.float32)] * ring,
            [pltpu.SemaphoreType.DMA] * ring,
            [pltpu.SemaphoreType.DMA] * ring,
        ],
        compiler_params=pltpu.CompilerParams(
            use_tc_tiling_on_sc=True, needs_layout_passes=False
        ),
    )
    def transpose_kernel(tt_hbm, out_hbm, in_v, out_v, isems, osems):
        wid = lax.axis_index("s") * nc + lax.axis_index("c")
        lane = lax.iota(jnp.int32, 16)

        def in_desc(i, b):
            blk = wid + i * nw
            return pltpu.make_async_copy(
                tt_hbm.at[:, pl.ds(blk * blk_rows, blk_rows)], in_v[b], isems[b]
            )

        def out_desc(i, b):
            blk = wid + i * nw
            return pltpu.make_async_copy(
                out_v[b],
                out_hbm.at[pl.ds(blk * blk_rows * d, blk_rows * d)],
                osems[b],
            )

        cbn = d // 16

        def transpose(b):
            @plsc.parallel_loop(0, (blk_rows // 16) * cbn, 1, unroll=8)
            def _(k):
                lg = k >> (cbn.bit_length() - 1)
                cb = k & (cbn - 1)
                r_vec = lg * 16 + lane
                rd = r_vec * d + cb * 16
                for c0 in range(16):
                    cperm = (lane + c0) & 15
                    c_vec = cperm + cb * 16
                    vals = plsc.load_gather(in_v[b], [c_vec, r_vec])
                    plsc.store_scatter(out_v[b], [rd + cperm], vals)

        for b in range(ring):

            @pl.when(wid + b * nw < n_blks)
            def _(b=b):
                in_desc(b, b).start()

        def step(i, b):
            blk = wid + i * nw

            @pl.when(blk < n_blks)
            def _():
                @pl.when(i >= ring)
                def _():
                    out_desc(i - ring, b).wait()

                in_desc(i, b).wait()
                transpose(b)
                out_desc(i, b).start()

                @pl.when(blk + ring * nw < n_blks)
                def _():
                    in_desc(i + ring, b).start()

        def body(g, carry):
            for b in range(ring):
                step(g * ring + b, b)
            return carry

        lax.fori_loop(0, (iters + ring - 1) // ring, body, 0)

        i_max = (n_blks - 1 - wid) // nw
        for b in range(ring):
            i_b = i_max - lax.rem(i_max - b + 2 * ring, ring)

            @pl.when(i_b >= 0)
            def _(b=b, i_b=i_b):
                out_desc(i_b, b).wait()

    return transpose_kernel


@functools.lru_cache(maxsize=None)
def _make_gather(n_rows, d, v_main, n_tail, chunk):
    mesh, nc, ns = _mesh_and_info()
    nw = nc * ns
    assert n_rows % nw == 0
    b_per_w = n_rows // nw
    assert b_per_w % chunk == 0 and chunk % 16 == 0
    n_chunks = b_per_w // chunk
    assert n_chunks % _NBUF == 0 and n_chunks >= 2 * _NBUF
    n_groups = n_chunks // _NBUF

    @functools.partial(
        pl.kernel,
        mesh=mesh,
        out_type=jax.ShapeDtypeStruct((n_rows, d), jnp.float32),
        scratch_types=[
            pltpu.VMEM((b_per_w,), jnp.int32),
            pltpu.VMEM((b_per_w,), jnp.int32),
            pltpu.VMEM((n_tail, d), jnp.float32),
            pltpu.VMEM((_NBUF, chunk, d), jnp.float32),
            [pltpu.SemaphoreType.DMA] * _NBUF,
            [pltpu.SemaphoreType.DMA] * _NBUF,
        ],
        compiler_params=pltpu.CompilerParams(
            use_tc_tiling_on_sc=False, needs_layout_passes=False
        ),
    )
    def gather_kernel(
        idx_hbm, table_hbm, tail_hbm, out_hbm, idx_v, idx_c, tail_v, rows_v,
        gsems, ssems,
    ):
        wid = lax.axis_index("s") * nc + lax.axis_index("c")
        base = wid * b_per_w
        pltpu.sync_copy(idx_hbm.at[pl.ds(base, b_per_w)], idx_v)
        pltpu.sync_copy(tail_hbm, tail_v)

        def clamp(g, carry):
            x = idx_v[pl.ds(g * 16, 16)]
            idx_c[pl.ds(g * 16, 16)] = jnp.minimum(x, v_main - 1)
            return carry

        lax.fori_loop(0, b_per_w // 16, clamp, 0)

        def g_desc(i, b):
            return pltpu.make_async_copy(
                table_hbm.at[idx_c.at[pl.ds(i * chunk, chunk)]],
                rows_v.at[b],
                gsems[b],
            )

        def s_desc(i, b):
            return pltpu.make_async_copy(
                rows_v.at[b],
                out_hbm.at[pl.ds(base + i * chunk, chunk)],
                ssems[b],
            )

        def patch_tail(j, b):
            off = j * chunk

            def has_tail(g, acc):
                x = idx_v[pl.ds(off + g * 16, 16)]
                return jnp.maximum(acc, jnp.max(x))

            mx = lax.fori_loop(0, chunk // 16, has_tail, jnp.int32(0))

            @pl.when(mx >= v_main)
            def _():
                lane = lax.iota(jnp.int32, 16)

                def fix(g, carry):
                    idx16 = idx_v[pl.ds(off + g * 16, 16)]
                    m = idx16 >= v_main
                    r = jnp.maximum(idx16 - v_main, 0)
                    jrow = lane + g * 16
                    for c in range(d):
                        cvec = jnp.full((16,), c, jnp.int32)
                        vals = plsc.load_gather(tail_v, [r, cvec], mask=m)
                        plsc.store_scatter(
                            rows_v.at[b], [jrow, cvec], vals, mask=m
                        )
                    return carry

                lax.fori_loop(0, chunk // 16, fix, 0)

        def step(j, b, wait_store, start_gather):
            if wait_store:
                s_desc(j - 2, (b + 2) % _NBUF).wait()
            if start_gather:
                g_desc(j + 2, (b + 2) % _NBUF).start()
            g_desc(j, b).wait()
            patch_tail(j, b)
            s_desc(j, b).start()

        g_desc(0, 0).start()
        g_desc(1, 1).start()
        for b in range(_NBUF):
            step(b, b, wait_store=b >= 2, start_gather=True)

        def body(g, carry):
            for b in range(_NBUF):
                step(g * _NBUF + b, b, wait_store=True, start_gather=True)
            return carry

        lax.fori_loop(1, n_groups - 1, body, 0)

        for b in range(_NBUF):
            j = (n_groups - 1) * _NBUF + b
            step(j, b, wait_store=True, start_gather=j + 2 < n_chunks)
        s_desc(n_chunks - 2, (n_chunks - 2) % _NBUF).wait()
        s_desc(n_chunks - 1, (n_chunks - 1) % _NBUF).wait()

    return gather_kernel


@functools.lru_cache(maxsize=None)
def _make_imagize(bsz, hsz, d):
    mesh, nc, ns = _mesh_and_info()
    nw = nc * ns
    nbt = bsz // 128
    n_units = hsz * nbt
    assert n_units % nw == 0
    iters = n_units // nw
    ring = 3
    unit = 128 * d
    cbn = d // 16

    @functools.partial(
        pl.kernel,
        mesh=mesh,
        out_type=jax.ShapeDtypeStruct((bsz * hsz * d,), jnp.float32),
        scratch_types=[
            [pltpu.VMEM((128, 1, d), jnp.float32)] * ring,
            [pltpu.VMEM((unit,), jnp.float32)] * ring,
            [pltpu.SemaphoreType.DMA] * ring,
            [pltpu.SemaphoreType.DMA] * ring,
        ],
        compiler_params=pltpu.CompilerParams(
            use_tc_tiling_on_sc=False, needs_layout_passes=False
        ),
    )
    def imagize_kernel(in_hbm, out_hbm, in_v, out_v, isems, osems):
        wid = lax.axis_index("s") * nc + lax.axis_index("c")
        lane = lax.iota(jnp.int32, 16)
        zero = jnp.full((16,), 0, jnp.int32)

        def in_desc(i, b):
            u = wid + i * nw
            h = u // nbt
            bt = u - h * nbt
            return pltpu.make_async_copy(
                in_hbm.at[pl.ds(bt * 128, 128), pl.ds(h, 1)], in_v[b], isems[b]
            )

        def start_out(i, b):
            u = wid + i * nw
            h = u // nbt
            bt = u - h * nbt
            for ct in range(d // 8):
                pltpu.make_async_copy(
                    out_v[b].at[pl.ds(ct * 1024, 1024)],
                    out_hbm.at[pl.ds((((h * 8 + ct) * nbt) + bt) * 1024, 1024)],
                    osems[b],
                ).start()

        def wait_out(i, b):
            u = wid + i * nw
            h = u // nbt
            bt = u - h * nbt
            for ct in range(d // 8):
                pltpu.make_async_copy(
                    out_v[b].at[pl.ds(ct * 1024, 1024)],
                    out_hbm.at[pl.ds((((h * 8 + ct) * nbt) + bt) * 1024, 1024)],
                    osems[b],
                ).wait()

        def rearrange(b):
            @plsc.parallel_loop(0, 8 * cbn, 1, unroll=8)
            def _(k):
                bg = k >> (cbn.bit_length() - 1)
                cb = k & (cbn - 1)
                bi = bg * 16 + lane
                sb = cb * 2048
                for c0 in range(16):
                    cperm = (lane + c0) & 15
                    c_vec = cperm + cb * 16
                    dstp = ((cperm >> 3) * 1024 + (cperm & 7) * 128) + sb
                    vals = plsc.load_gather(in_v[b], [bi, zero, c_vec])
                    plsc.store_scatter(out_v[b], [dstp + bi], vals)

        for b in range(ring):

            @pl.when(wid + b * nw < n_units)
            def _(b=b):
                in_desc(b, b).start()

        def step(i, b, dyn):
            if dyn:

                @pl.when(i >= ring)
                def _():
                    wait_out(i - ring, b)

            elif i >= ring:
                wait_out(i - ring, b)

            in_desc(i, b).wait()
            rearrange(b)
            start_out(i, b)

            if dyn:

                @pl.when(i + ring < iters)
                def _():
                    in_desc(i + ring, b).start()

            elif i + ring < iters:
                in_desc(i + ring, b).start()

        def body(g, carry):
            for b in range(ring):
                step(g * ring + b, b, dyn=True)
            return carry

        lax.fori_loop(0, iters // ring, body, 0)
        for j in range(iters - iters % ring, iters):
            step(j, j % ring, dyn=False)
        for j in range(iters - ring, iters):
            wait_out(j, j % ring)

    return imagize_kernel


def kernel(indices, table):
    b, h = indices.shape
    v, d = table.shape
    v_main = (v // 128) * 128
    n_tail = v - v_main
    idx_flat = indices.reshape(b * h).astype(jnp.int32)
    lin = _make_transpose(d, v_main)(table.T)
    tail = lax.slice(table, (v_main, 0), (v, d))
    out = _make_gather(b * h, d, v_main, n_tail, 320)(
        idx_flat, lin.reshape(v_main, d), tail
    )
    img = _make_imagize(b, h, d)(out.reshape(b, h, d))
    x = img.reshape(h, d // 8, b // 128, 8, 128)
    return x.transpose(2, 4, 0, 1, 3).reshape(b, h, d)
